# baseline (speedup 1.0000x reference)
import functools

import jax
import jax.numpy as jnp
from jax import lax
from jax.experimental import pallas as pl
from jax.experimental.pallas import tpu as pltpu
from jax.experimental.pallas import tpu_sc as plsc

_ATTR_INDEX = 8

_NC = 2
_NS = 16
_L = 16
_NW = _NC * _NS
_CH = 256
_NBUF = 3


def _make_sc_kernel(B, C, Bh, h):
    n = Bh // _NW
    nch = n // _CH
    h0 = h * Bh

    mesh = plsc.VectorSubcoreMesh(core_axis_name="c", subcore_axis_name="s")

    @functools.partial(
        pl.kernel,
        mesh=mesh,
        out_type=jax.ShapeDtypeStruct((Bh,), jnp.float32),
        compiler_params=pltpu.CompilerParams(needs_layout_passes=False),
        scratch_types=[
            pltpu.VMEM((_NBUF * _CH, C), jnp.float32),
            pltpu.VMEM((n,), jnp.float32),
            pltpu.VMEM((n,), jnp.float32),
            pltpu.SemaphoreType.DMA,
        ],
    )
    def k(zcol_hbm, a_hbm, out_hbm, abuf, cbuf, obuf, asem):
        wid = lax.axis_index("s") * _NC + lax.axis_index("c")
        base = wid * n

        iota = lax.iota(jnp.int32, _L)

        def a_copy(ch):
            return pltpu.make_async_copy(
                a_hbm.at[pl.ds(h0 + base + ch * _CH, _CH)],
                abuf.at[pl.ds((ch % _NBUF) * _CH, _CH)],
                asem,
            )

        for ch in range(min(_NBUF - 1, nch)):
            a_copy(ch).start()
        pltpu.sync_copy(zcol_hbm.at[pl.ds(base, n)], cbuf)

        for ch in range(nch):
            a_copy(ch).wait()
            if ch + _NBUF - 1 < nch:
                a_copy(ch + _NBUF - 1).start()
            par = (ch % _NBUF) * _CH

            def extract(j, carry, ch=ch, par=par):
                rows = par + j * _L + iota
                cols = cbuf[pl.ds(ch * _CH + j * _L, _L)].astype(jnp.int32)
                av = plsc.load_gather(abuf, [rows, cols])
                obuf[pl.ds(ch * _CH + j * _L, _L)] = av * jnp.float32(0.999)
                return carry

            lax.fori_loop(0, _CH // _L, extract, 0)

        pltpu.sync_copy(obuf, out_hbm.at[pl.ds(base, n)])

    return k


_H = 2


@jax.jit
def kernel(z, a):
    b, c = a.shape
    bh = b // _H
    parts = []
    for h in range(_H):
        zcol = z[h * bh:(h + 1) * bh, _ATTR_INDEX]
        parts.append(_make_sc_kernel(b, c, bh, h)(zcol, a))
    return jnp.concatenate(parts)
